# Initial kernel scaffold; baseline (speedup 1.0000x reference)
import jax
import jax.numpy as jnp
from jax.experimental import pallas as pl


def kernel(indices, R):
    raise NotImplementedError("write your pallas kernel here")


# speedup vs baseline: 5.1740x; 5.1740x over previous
import functools

import jax
import jax.numpy as jnp
from jax import lax
from jax.experimental import pallas as pl
from jax.experimental.pallas import tpu as pltpu
from jax.experimental.pallas import tpu_sc as plsc

_CHUNK = 128


@functools.cache
def _build(B, D, NC, NS, n_ch):
    NW = NC * NS
    mesh = plsc.VectorSubcoreMesh(core_axis_name="c", subcore_axis_name="s")

    @functools.partial(
        pl.kernel,
        mesh=mesh,
        compiler_params=pltpu.CompilerParams(use_tc_tiling_on_sc=False),
        out_type=jax.ShapeDtypeStruct((B, D), jnp.float32),
        scratch_types=[
            pltpu.VMEM((n_ch, _CHUNK), jnp.int32),
            pltpu.VMEM((_CHUNK, D), jnp.float32),
            pltpu.SemaphoreType.DMA,
        ],
    )
    def gather(idx_hbm, table_hbm, out_hbm, idx_v, rows_v, sem):
        wid = lax.axis_index("s") * NC + lax.axis_index("c")
        base = wid * (n_ch * _CHUNK)
        pltpu.sync_copy(idx_hbm.at[wid], idx_v)

        def body(j, carry):
            pltpu.async_copy(table_hbm.at[idx_v.at[j]], rows_v, sem).wait()
            pltpu.sync_copy(rows_v, out_hbm.at[pl.ds(base + j * _CHUNK, _CHUNK)])
            return carry

        lax.fori_loop(0, n_ch, body, 0)

    return gather


def kernel(indices, R):
    B0, K = indices.shape
    V, D = R.shape
    idx = indices.reshape(-1).astype(jnp.int32)
    B = idx.shape[0]
    info = plsc.get_sparse_core_info()
    NC, NS = info.num_cores, info.num_subcores
    NW = NC * NS
    assert B % (NW * _CHUNK) == 0
    n_ch = B // (NW * _CHUNK)
    idx3 = idx.reshape(NW, n_ch, _CHUNK)
    out = _build(B, D, NC, NS, n_ch)(idx3, R)
    return out.reshape(B0, K, D)


# speedup vs baseline: 6.0846x; 1.1760x over previous
import functools

import jax
import jax.numpy as jnp
from jax import lax
from jax.experimental import pallas as pl
from jax.experimental.pallas import tpu as pltpu
from jax.experimental.pallas import tpu_sc as plsc

_CHUNK = 128
_NB = 4


@functools.cache
def _build(B, D, NC, NS, n_ch):
    NW = NC * NS
    n_sup = n_ch // _NB
    SUP = _NB * _CHUNK
    mesh = plsc.VectorSubcoreMesh(core_axis_name="c", subcore_axis_name="s")

    @functools.partial(
        pl.kernel,
        mesh=mesh,
        compiler_params=pltpu.CompilerParams(use_tc_tiling_on_sc=False),
        out_type=jax.ShapeDtypeStruct((B, D), jnp.float32),
        scratch_types=[
            pltpu.VMEM((n_ch, _CHUNK), jnp.int32),
            pltpu.VMEM((SUP, D), jnp.float32),
            pltpu.VMEM((SUP, D), jnp.float32),
            pltpu.SemaphoreType.DMA,
            pltpu.SemaphoreType.DMA,
            pltpu.SemaphoreType.DMA,
        ],
    )
    def gather(idx_hbm, table_hbm, out_hbm, idx_v, rows0, rows1, gsem, osem0, osem1):
        wid = lax.axis_index("s") * NC + lax.axis_index("c")
        base = wid * (n_ch * _CHUNK)
        pltpu.sync_copy(idx_hbm.at[wid], idx_v)
        rows = (rows0, rows1)
        osem = (osem0, osem1)

        def fire(s, p):
            for b in range(_NB):
                pltpu.async_copy(
                    table_hbm.at[idx_v.at[s * _NB + b]],
                    rows[p].at[pl.ds(b * _CHUNK, _CHUNK)],
                    gsem,
                )

        def drain_gathers(p):
            pltpu.make_async_copy(table_hbm.at[pl.ds(0, SUP)], rows[p], gsem).wait()

        def start_out(s, p):
            pltpu.async_copy(rows[p], out_hbm.at[pl.ds(base + s * SUP, SUP)], osem[p])

        def drain_out(s, p):
            pltpu.make_async_copy(
                rows[p], out_hbm.at[pl.ds(base + s * SUP, SUP)], osem[p]
            ).wait()

        fire(0, 0)
        for s in range(n_sup):
            p = s % 2
            drain_gathers(p)
            start_out(s, p)
            if s + 1 < n_sup:
                if s >= 1:
                    drain_out(s - 1, 1 - p)
                fire(s + 1, 1 - p)
        drain_out(n_sup - 2, n_sup % 2)
        drain_out(n_sup - 1, (n_sup - 1) % 2)

    return gather


def kernel(indices, R):
    B0, K = indices.shape
    V, D = R.shape
    idx = indices.reshape(-1).astype(jnp.int32)
    B = idx.shape[0]
    info = plsc.get_sparse_core_info()
    NC, NS = info.num_cores, info.num_subcores
    NW = NC * NS
    assert B % (NW * _CHUNK * _NB) == 0
    n_ch = B // (NW * _CHUNK)
    idx3 = idx.reshape(NW, n_ch, _CHUNK)
    out = _build(B, D, NC, NS, n_ch)(idx3, R)
    return out.reshape(B0, K, D)
